# initial kernel scaffold (unmeasured)
import jax
import jax.numpy as jnp
from jax import lax
from jax.experimental import pallas as pl
from jax.experimental.pallas import tpu as pltpu

N_DEV = 8
EPS = 1e-5


def kernel(x, gamma):
    m, n_per = x.shape
    n_global = n_per * N_DEV

    def body(x_ref, g_ref, out_ref, gather_ref, send_sems, recv_sems):
        my = lax.axis_index("i")

        barrier_sem = pltpu.get_barrier_semaphore()
        for d in range(1, N_DEV):
            peer = lax.rem(my + d, N_DEV)
            pl.semaphore_signal(
                barrier_sem, inc=1,
                device_id=(peer,), device_id_type=pl.DeviceIdType.MESH,
            )
        pl.semaphore_wait(barrier_sem, N_DEV - 1)

        xf = x_ref[:, :].astype(jnp.float32)
        xsq = xf * xf
        ones_row = jnp.ones((1, n_per), jnp.float32)
        p_row = lax.dot_general(
            ones_row, xsq,
            (((1,), (1,)), ((), ())),
            preferred_element_type=jnp.float32,
            precision=lax.Precision.HIGHEST,
        )
        gather_ref[pl.ds(my, 1), :] = p_row

        for d in range(1, N_DEV):
            peer = lax.rem(my + d, N_DEV)
            rdma = pltpu.make_async_remote_copy(
                src_ref=gather_ref.at[pl.ds(my, 1)],
                dst_ref=gather_ref.at[pl.ds(my, 1)],
                send_sem=send_sems.at[d - 1],
                recv_sem=recv_sems.at[d - 1],
                device_id=(peer,),
                device_id_type=pl.DeviceIdType.MESH,
            )
            rdma.start()

        for d in range(1, N_DEV):
            peer = lax.rem(my + d, N_DEV)
            wait = pltpu.make_async_remote_copy(
                src_ref=gather_ref.at[pl.ds(my, 1)],
                dst_ref=gather_ref.at[pl.ds(my, 1)],
                send_sem=send_sems.at[d - 1],
                recv_sem=recv_sems.at[d - 1],
                device_id=(peer,),
                device_id_type=pl.DeviceIdType.MESH,
            )
            wait.wait_recv()
            wait.wait_send()

        ones_col = jnp.ones((N_DEV, 1), jnp.float32)
        tot_col = lax.dot_general(
            gather_ref[:, :], ones_col,
            (((0,), (0,)), ((), ())),
            preferred_element_type=jnp.float32,
            precision=lax.Precision.HIGHEST,
        )
        inv_col = lax.rsqrt(tot_col / n_global + EPS)

        g_row = g_ref[:].astype(jnp.float32)[None, :]
        out_ref[:, :] = (xf * inv_col * g_row).astype(out_ref.dtype)

    return pl.pallas_call(
        body,
        out_shape=jax.ShapeDtypeStruct((m, n_per), jnp.bfloat16),
        in_specs=[
            pl.BlockSpec(memory_space=pltpu.VMEM),
            pl.BlockSpec(memory_space=pltpu.VMEM),
        ],
        out_specs=pl.BlockSpec(memory_space=pltpu.VMEM),
        scratch_shapes=[
            pltpu.VMEM((N_DEV, m), jnp.float32),
            pltpu.SemaphoreType.DMA((N_DEV - 1,)),
            pltpu.SemaphoreType.DMA((N_DEV - 1,)),
        ],
        compiler_params=pltpu.CompilerParams(collective_id=0),
    )(x, gamma)


# baseline (device time: 40758 ns/iter reference)
import jax
import jax.numpy as jnp
from jax import lax
from jax.experimental import pallas as pl
from jax.experimental.pallas import tpu as pltpu

N_DEV = 8
EPS = 1e-5


def kernel(x, gamma):
    m, n_per = x.shape
    n_global = n_per * N_DEV

    def body(x_ref, g_ref, out_ref, gather_ref, send_sems, recv_sems):
        my = lax.axis_index("i")

        barrier_sem = pltpu.get_barrier_semaphore()
        for d in range(1, N_DEV):
            peer = lax.rem(my + d, N_DEV)
            pl.semaphore_signal(
                barrier_sem, inc=1,
                device_id=(peer,), device_id_type=pl.DeviceIdType.MESH,
            )
        pl.semaphore_wait(barrier_sem, N_DEV - 1)

        xf = x_ref[:, :].astype(jnp.float32)
        xsq = xf * xf
        ones_row = jnp.ones((1, n_per), jnp.float32)
        p_row = lax.dot_general(
            ones_row, xsq,
            (((1,), (1,)), ((), ())),
            preferred_element_type=jnp.float32,
            precision=lax.Precision.HIGHEST,
        )
        gather_ref[pl.ds(my, 1), :] = p_row

        for d in range(1, N_DEV):
            peer = lax.rem(my + d, N_DEV)
            rdma = pltpu.make_async_remote_copy(
                src_ref=gather_ref.at[pl.ds(my, 1)],
                dst_ref=gather_ref.at[pl.ds(my, 1)],
                send_sem=send_sems.at[d - 1],
                recv_sem=recv_sems.at[d - 1],
                device_id=(peer,),
                device_id_type=pl.DeviceIdType.MESH,
            )
            rdma.start()

        for d in range(1, N_DEV):
            peer = lax.rem(my + d, N_DEV)
            wait = pltpu.make_async_remote_copy(
                src_ref=gather_ref.at[pl.ds(my, 1)],
                dst_ref=gather_ref.at[pl.ds(my, 1)],
                send_sem=send_sems.at[d - 1],
                recv_sem=recv_sems.at[d - 1],
                device_id=(peer,),
                device_id_type=pl.DeviceIdType.MESH,
            )
            wait.wait_recv()
            wait.wait_send()

        ones_col = jnp.ones((N_DEV, 1), jnp.float32)
        tot_col = lax.dot_general(
            gather_ref[:, :], ones_col,
            (((0,), (0,)), ((), ())),
            preferred_element_type=jnp.float32,
            precision=lax.Precision.HIGHEST,
        )
        inv_col = lax.rsqrt(tot_col / n_global + EPS)

        g_row = g_ref[:].astype(jnp.float32)[None, :]
        out_ref[:, :] = (xf * inv_col * g_row).astype(out_ref.dtype)

    return pl.pallas_call(
        body,
        out_shape=jax.ShapeDtypeStruct((m, n_per), jnp.bfloat16),
        in_specs=[
            pl.BlockSpec(memory_space=pltpu.VMEM),
            pl.BlockSpec(memory_space=pltpu.VMEM),
        ],
        out_specs=pl.BlockSpec(memory_space=pltpu.VMEM),
        scratch_shapes=[
            pltpu.VMEM((N_DEV, m), jnp.float32),
            pltpu.SemaphoreType.DMA((N_DEV - 1,)),
            pltpu.SemaphoreType.DMA((N_DEV - 1,)),
        ],
        compiler_params=pltpu.CompilerParams(
            collective_id=0,
            vmem_limit_bytes=60 * 1024 * 1024,
        ),
    )(x, gamma)


# device time: 26064 ns/iter; 1.5638x vs baseline; 1.5638x over previous
import jax
import jax.numpy as jnp
from jax import lax
from jax.experimental import pallas as pl
from jax.experimental.pallas import tpu as pltpu

N_DEV = 8
EPS = 1e-5


def kernel(x, gamma):
    m, n_per = x.shape
    n_global = n_per * N_DEV

    def body(x_ref, g_ref, out_ref, gather_ref, send_sems, recv_sems):
        my = lax.axis_index("i")

        barrier_sem = pltpu.get_barrier_semaphore()
        for d in range(1, N_DEV):
            peer = lax.rem(my + d, N_DEV)
            pl.semaphore_signal(
                barrier_sem, inc=1,
                device_id=(peer,), device_id_type=pl.DeviceIdType.MESH,
            )

        xf = x_ref[:, :].astype(jnp.float32)
        p_col = jnp.sum(xf * xf, axis=1, keepdims=True)
        gather_ref[pl.ds(my, 1), :] = lax.transpose(p_col, (1, 0))

        pl.semaphore_wait(barrier_sem, N_DEV - 1)

        for d in range(1, N_DEV):
            peer = lax.rem(my + d, N_DEV)
            rdma = pltpu.make_async_remote_copy(
                src_ref=gather_ref.at[pl.ds(my, 1)],
                dst_ref=gather_ref.at[pl.ds(my, 1)],
                send_sem=send_sems.at[d - 1],
                recv_sem=recv_sems.at[d - 1],
                device_id=(peer,),
                device_id_type=pl.DeviceIdType.MESH,
            )
            rdma.start()

        for d in range(1, N_DEV):
            peer = lax.rem(my + d, N_DEV)
            wait = pltpu.make_async_remote_copy(
                src_ref=gather_ref.at[pl.ds(my, 1)],
                dst_ref=gather_ref.at[pl.ds(my, 1)],
                send_sem=send_sems.at[d - 1],
                recv_sem=recv_sems.at[d - 1],
                device_id=(peer,),
                device_id_type=pl.DeviceIdType.MESH,
            )
            wait.wait_recv()
            wait.wait_send()

        tot_row = jnp.sum(gather_ref[:, :], axis=0, keepdims=True)
        tot_col = lax.transpose(tot_row, (1, 0))
        inv_col = lax.rsqrt(tot_col / n_global + EPS)
        g_row = g_ref[:].astype(jnp.float32)[None, :]
        out_ref[:, :] = (xf * inv_col * g_row).astype(out_ref.dtype)

    return pl.pallas_call(
        body,
        out_shape=jax.ShapeDtypeStruct((m, n_per), jnp.bfloat16),
        in_specs=[
            pl.BlockSpec(memory_space=pltpu.VMEM),
            pl.BlockSpec(memory_space=pltpu.VMEM),
        ],
        out_specs=pl.BlockSpec(memory_space=pltpu.VMEM),
        scratch_shapes=[
            pltpu.VMEM((N_DEV, m), jnp.float32),
            pltpu.SemaphoreType.DMA((N_DEV - 1,)),
            pltpu.SemaphoreType.DMA((N_DEV - 1,)),
        ],
        compiler_params=pltpu.CompilerParams(
            collective_id=0,
            vmem_limit_bytes=60 * 1024 * 1024,
        ),
    )(x, gamma)


# device time: 26014 ns/iter; 1.5668x vs baseline; 1.0019x over previous
import jax
import jax.numpy as jnp
from jax import lax
from jax.experimental import pallas as pl
from jax.experimental.pallas import tpu as pltpu

N_DEV = 8
EPS = 1e-5
N_CHUNK = 4


def kernel(x, gamma):
    m, n_per = x.shape
    n_global = n_per * N_DEV
    mc = m // N_CHUNK

    def body(x_ref, g_ref, out_hbm, out_vmem, gather_ref,
             send_sems, recv_sems, copy_sems):
        my = lax.axis_index("i")

        barrier_sem = pltpu.get_barrier_semaphore()
        for d in range(1, N_DEV):
            peer = lax.rem(my + d, N_DEV)
            pl.semaphore_signal(
                barrier_sem, inc=1,
                device_id=(peer,), device_id_type=pl.DeviceIdType.MESH,
            )

        xf = x_ref[:, :].astype(jnp.float32)
        p_col = jnp.sum(xf * xf, axis=1, keepdims=True)
        gather_ref[pl.ds(my, 1), :] = lax.transpose(p_col, (1, 0))

        pl.semaphore_wait(barrier_sem, N_DEV - 1)

        for d in range(1, N_DEV):
            peer = lax.rem(my + d, N_DEV)
            rdma = pltpu.make_async_remote_copy(
                src_ref=gather_ref.at[pl.ds(my, 1)],
                dst_ref=gather_ref.at[pl.ds(my, 1)],
                send_sem=send_sems.at[d - 1],
                recv_sem=recv_sems.at[d - 1],
                device_id=(peer,),
                device_id_type=pl.DeviceIdType.MESH,
            )
            rdma.start()

        for d in range(1, N_DEV):
            peer = lax.rem(my + d, N_DEV)
            wait = pltpu.make_async_remote_copy(
                src_ref=gather_ref.at[pl.ds(my, 1)],
                dst_ref=gather_ref.at[pl.ds(my, 1)],
                send_sem=send_sems.at[d - 1],
                recv_sem=recv_sems.at[d - 1],
                device_id=(peer,),
                device_id_type=pl.DeviceIdType.MESH,
            )
            wait.wait_recv()
            wait.wait_send()

        tot_row = jnp.sum(gather_ref[:, :], axis=0, keepdims=True)
        tot_col = lax.transpose(tot_row, (1, 0))
        inv_col = lax.rsqrt(tot_col / n_global + EPS)
        g_row = g_ref[:].astype(jnp.float32)[None, :]

        copies = []
        for c in range(N_CHUNK):
            rows = pl.ds(c * mc, mc)
            out_vmem[rows, :] = (
                xf[c * mc:(c + 1) * mc, :] * inv_col[c * mc:(c + 1) * mc, :]
                * g_row
            ).astype(out_vmem.dtype)
            cp = pltpu.make_async_copy(
                out_vmem.at[rows], out_hbm.at[rows], copy_sems.at[c]
            )
            cp.start()
            copies.append(cp)
        for cp in copies:
            cp.wait()

    return pl.pallas_call(
        body,
        out_shape=jax.ShapeDtypeStruct((m, n_per), jnp.bfloat16),
        in_specs=[
            pl.BlockSpec(memory_space=pltpu.VMEM),
            pl.BlockSpec(memory_space=pltpu.VMEM),
        ],
        out_specs=pl.BlockSpec(memory_space=pltpu.MemorySpace.HBM),
        scratch_shapes=[
            pltpu.VMEM((m, n_per), jnp.bfloat16),
            pltpu.VMEM((N_DEV, m), jnp.float32),
            pltpu.SemaphoreType.DMA((N_DEV - 1,)),
            pltpu.SemaphoreType.DMA((N_DEV - 1,)),
            pltpu.SemaphoreType.DMA((N_CHUNK,)),
        ],
        compiler_params=pltpu.CompilerParams(
            collective_id=0,
            vmem_limit_bytes=60 * 1024 * 1024,
        ),
    )(x, gamma)
